# baseline (speedup 1.0000x reference)
import functools

import jax
import jax.numpy as jnp
from jax import lax
from jax.experimental import pallas as pl
from jax.experimental.pallas import tpu as pltpu
from jax.experimental.pallas import tpu_sc as plsc

_NUM_PLAYLISTS = 2000
_NUM_TRACKS = 8000
_D = 128
_N = _NUM_PLAYLISTS + _NUM_TRACKS
_N_PAD = 10240
_E = 320000
_NLAYERS = 3

_CH = 32
_NWORKERS = 32
_NCHT = 324
_E_PAD = _NWORKERS * _NCHT * _CH
_ROWS_PER_TILE = _N_PAD // 16
_LANES = 16
_NBUF = 6
_NOUT = 3

_mesh = plsc.VectorSubcoreMesh(core_axis_name="c", subcore_axis_name="s")

_scratch = (
    [pltpu.VMEM((_CH,), jnp.int32) for _ in range(_NBUF)]
    + [pltpu.VMEM((_CH,), jnp.int32) for _ in range(_NBUF)]
    + [pltpu.VMEM((_CH,), jnp.float32) for _ in range(_NBUF)]
    + [pltpu.VMEM((_CH, _D // 2), jnp.int32) for _ in range(_NBUF)]
    + [pltpu.VMEM((_CH, _D), jnp.float32) for _ in range(_NOUT)]
    + [pltpu.VMEM_SHARED((_N_PAD, _D), jnp.float32)]
    + [pltpu.SemaphoreType.DMA for _ in range(3 * _NBUF + _NOUT)]
)


@functools.partial(
    pl.kernel,
    out_type=[
        jax.ShapeDtypeStruct((_N_PAD, _D), jnp.float32),
        jax.ShapeDtypeStruct((_N_PAD, _D), jnp.float32),
    ],
    mesh=_mesh,
    scratch_types=_scratch,
    compiler_params=pltpu.CompilerParams(
        needs_layout_passes=False, use_tc_tiling_on_sc=False),
)
def _sc_layer(tbl, src1, dst1, w1, out0, out1, *refs):
    o = 0
    srcb = refs[o:o + _NBUF]; o += _NBUF
    dstb = refs[o:o + _NBUF]; o += _NBUF
    wb = refs[o:o + _NBUF]; o += _NBUF
    rin = refs[o:o + _NBUF]; o += _NBUF
    rout = refs[o:o + _NOUT]; o += _NOUT
    acc = refs[o]; o += 1
    esemS = refs[o:o + _NBUF]; o += _NBUF
    esemD = refs[o:o + _NBUF]; o += _NBUF
    gsem = refs[o:o + _NBUF]; o += _NBUF
    ssem = refs[o:o + _NOUT]; o += _NOUT

    c = lax.axis_index("c")
    s = lax.axis_index("s")
    wid = s * 2 + c
    ebase = wid * _NCHT * _CH

    def _zero_row(i, _):
        for d in range(_D // _LANES):
            rout[0][i, pl.ds(d * _LANES, _LANES)] = jnp.zeros((_LANES,), jnp.float32)
        return 0
    lax.fori_loop(0, _CH, _zero_row, 0)
    rbase = s * _ROWS_PER_TILE
    for k in range(_ROWS_PER_TILE // _CH):
        pltpu.sync_copy(rout[0], acc.at[pl.ds(rbase + k * _CH, _CH)])
    plsc.subcore_barrier()

    def _ef_srcw(g, b):
        cb = ebase + g * _CH
        pltpu.async_copy(src1.at[pl.ds(cb, _CH)], srcb[b], esemS[b])
        pltpu.async_copy(w1.at[pl.ds(cb, _CH)], wb[b], esemS[b])

    def _efwait_srcw(g, b):
        cb = ebase + g * _CH
        pltpu.make_async_copy(src1.at[pl.ds(cb, _CH)], srcb[b], esemS[b]).wait()
        pltpu.make_async_copy(w1.at[pl.ds(cb, _CH)], wb[b], esemS[b]).wait()

    def _ef_dst(g, b):
        cb = ebase + g * _CH
        pltpu.async_copy(dst1.at[pl.ds(cb, _CH)], dstb[b], esemD[b])

    def _efwait_dst(g, b):
        cb = ebase + g * _CH
        pltpu.make_async_copy(dst1.at[pl.ds(cb, _CH)], dstb[b], esemD[b]).wait()

    _MASK = jnp.full((_LANES,), -65536, jnp.int32)
    _SH = jnp.full((_LANES,), 16, jnp.int32)

    def _scale(b, ob):
        def body(eg, _):
            wvec = wb[b][pl.ds(eg * _LANES, _LANES)]
            for j in range(_LANES):
                wsc = wvec[j]
                e = eg * _LANES + j
                for m in range(_D // 32):
                    wi = rin[b][e, pl.ds(m * _LANES, _LANES)]
                    lo = lax.bitcast_convert_type(lax.shift_left(wi, _SH), jnp.float32)
                    hi = lax.bitcast_convert_type(lax.bitwise_and(wi, _MASK), jnp.float32)
                    rout[ob][e, pl.ds(32 * m, _LANES)] = lo * wsc
                    rout[ob][e, pl.ds(32 * m + 16, _LANES)] = hi * wsc
            return 0
        lax.fori_loop(0, _CH // _LANES, body, 0)

    for g in range(4):
        _ef_srcw(g, g)
    for g in range(3):
        _ef_dst(g, g)
    for g in range(3):
        _efwait_srcw(g, g)
        pltpu.async_copy(tbl.at[srcb[g]], rin[g], gsem[g])

    def _iter(i, _):
        for b in range(_NBUF):
            g = i * _NBUF + b
            ob = b % _NOUT
            pltpu.make_async_copy(tbl.at[srcb[b]], rin[b], gsem[b]).wait()

            bm3 = (b - 3) % _NBUF
            obm3 = (b - 3) % _NOUT

            @pl.when(g >= 3)
            def _():
                pltpu.make_async_copy(rout[obm3], acc.at[dstb[bm3]], ssem[obm3]).wait()

            _scale(b, ob)
            _efwait_dst(g, b)
            pltpu.async_copy(rout[ob], acc.at[dstb[b]], ssem[ob], add=True)

            bp3 = (b + 3) % _NBUF
            bp4 = (b + 4) % _NBUF

            @pl.when(g + 3 < _NCHT)
            def _():
                _ef_dst(g + 3, bp3)

            @pl.when(g + 4 < _NCHT)
            def _():
                _ef_srcw(g + 4, bp4)

            @pl.when(g + 3 < _NCHT)
            def _():
                _efwait_srcw(g + 3, bp3)
                pltpu.async_copy(tbl.at[srcb[bp3]], rin[bp3], gsem[bp3])
        return 0
    lax.fori_loop(0, _NCHT // _NBUF, _iter, 0)
    for t in (3, 2, 1):
        gl = _NCHT - t
        pltpu.make_async_copy(rout[gl % _NOUT], acc.at[dstb[gl % _NBUF]], ssem[gl % _NOUT]).wait()
    plsc.subcore_barrier()

    for k in range(_ROWS_PER_TILE // 128):
        sl = pl.ds(rbase + k * 128, 128)

        @pl.when(c == 0)
        def _():
            pltpu.sync_copy(acc.at[sl], out0.at[sl])

        @pl.when(c == 1)
        def _():
            pltpu.sync_copy(acc.at[sl], out1.at[sl])


_ROWS_PER_WORKER = _N_PAD // _NWORKERS
_CCH = 64

_cscratch = (
    [pltpu.VMEM((_CCH, _D), jnp.float32) for _ in range(3)]
    + [pltpu.VMEM((_CCH, _D // 2), jnp.int32)]
)

def _mk_rnd16():
    bias = jnp.full((_LANES,), 0x7FFF, jnp.int32)
    one = jnp.full((_LANES,), 1, jnp.int32)
    sh16 = jnp.full((_LANES,), 16, jnp.int32)

    def rnd16(x):
        i = lax.bitcast_convert_type(x, jnp.int32)
        lsb = lax.bitwise_and(lax.shift_right_logical(i, sh16), one)
        return lax.add(lax.add(i, bias), lsb)
    return rnd16, sh16


@functools.partial(
    pl.kernel,
    out_type=[
        jax.ShapeDtypeStruct((_N_PAD, _D // 2), jnp.int32),
        jax.ShapeDtypeStruct((_N_PAD, _D), jnp.float32),
    ],
    mesh=_mesh,
    scratch_types=_cscratch,
    compiler_params=pltpu.CompilerParams(
        needs_layout_passes=False, use_tc_tiling_on_sc=False),
)
def _sc_combine(pa, pb, run, tbl_out, run_out, bufa, bufb, bufr, buft):
    c = lax.axis_index("c")
    s = lax.axis_index("s")
    wid = s * 2 + c
    rb = wid * _ROWS_PER_WORKER
    _rnd16, _R_SH16 = _mk_rnd16()
    _R_MASKHI = jnp.full((_LANES,), -65536, jnp.int32)
    _R_MASKLO = jnp.full((_LANES,), 0xFFFF, jnp.int32)
    for k in range(_ROWS_PER_WORKER // _CCH):
        sl = pl.ds(rb + k * _CCH, _CCH)
        pltpu.sync_copy(pa.at[sl], bufa)
        pltpu.sync_copy(pb.at[sl], bufb)
        pltpu.sync_copy(run.at[sl], bufr)

        def _row(i, _):
            for m in range(_D // 32):
                lo = bufa[i, pl.ds(32 * m, _LANES)] + bufb[i, pl.ds(32 * m, _LANES)]
                hi = bufa[i, pl.ds(32 * m + 16, _LANES)] + bufb[i, pl.ds(32 * m + 16, _LANES)]
                bufr[i, pl.ds(32 * m, _LANES)] = bufr[i, pl.ds(32 * m, _LANES)] + lo
                bufr[i, pl.ds(32 * m + 16, _LANES)] = bufr[i, pl.ds(32 * m + 16, _LANES)] + hi
                wlo = lax.bitwise_and(lax.shift_right_logical(_rnd16(lo), _R_SH16), _R_MASKLO)
                whi = lax.bitwise_and(_rnd16(hi), _R_MASKHI)
                buft[i, pl.ds(_LANES * m, _LANES)] = lax.bitwise_or(wlo, whi)
            return 0
        lax.fori_loop(0, _CCH, _row, 0)
        pltpu.sync_copy(bufr, run_out.at[sl])
        pltpu.sync_copy(buft, tbl_out.at[sl])


@functools.partial(
    pl.kernel,
    out_type=jax.ShapeDtypeStruct((_N_PAD, _D), jnp.float32),
    mesh=_mesh,
    scratch_types=[pltpu.VMEM((_CCH, _D), jnp.float32) for _ in range(3)],
    compiler_params=pltpu.CompilerParams(
        needs_layout_passes=False, use_tc_tiling_on_sc=False),
)
def _sc_final(pa, pb, run, out, bufa, bufb, bufr):
    c = lax.axis_index("c")
    s = lax.axis_index("s")
    wid = s * 2 + c
    rb = wid * _ROWS_PER_WORKER
    _R_QUART = jnp.full((_LANES,), 0.25, jnp.float32)
    for k in range(_ROWS_PER_WORKER // _CCH):
        sl = pl.ds(rb + k * _CCH, _CCH)
        pltpu.sync_copy(pa.at[sl], bufa)
        pltpu.sync_copy(pb.at[sl], bufb)
        pltpu.sync_copy(run.at[sl], bufr)

        def _row(i, _):
            for d in range(_D // _LANES):
                dsl = pl.ds(d * _LANES, _LANES)
                bufr[i, dsl] = (bufr[i, dsl] + bufa[i, dsl] + bufb[i, dsl]) * _R_QUART
            return 0
        lax.fori_loop(0, _CCH, _row, 0)
        pltpu.sync_copy(bufr, out.at[sl])


def _pack_table(emb):
    xp = emb.reshape(_N_PAD, 4, 2, 16).swapaxes(2, 3).reshape(_N_PAD, _D)
    b16 = xp.astype(jnp.bfloat16)
    return jax.lax.bitcast_convert_type(b16.reshape(_N_PAD, _D // 2, 2), jnp.int32)


def kernel(playlist_weight, track_weight, edge_index, edge_weight):
    emb0 = jnp.concatenate([playlist_weight, track_weight], axis=0)
    emb0 = jnp.pad(emb0, ((0, _N_PAD - _N), (0, 0)))
    src = edge_index[0].astype(jnp.int32)
    dst = edge_index[1].astype(jnp.int32)
    w = edge_weight.astype(jnp.float32)
    pad = _E_PAD - _E
    fill = _N + jnp.arange(pad, dtype=jnp.int32) % (_N_PAD - _N)
    src1 = jnp.concatenate([src, fill])
    dst1 = jnp.concatenate([dst, fill])
    w1 = jnp.concatenate([w, jnp.zeros((pad,), jnp.float32)])

    tbl = _pack_table(emb0)
    run = emb0
    final = None
    for layer in range(_NLAYERS):
        pa, pb = _sc_layer(tbl, src1, dst1, w1)
        if layer < _NLAYERS - 1:
            tbl, run = _sc_combine(pa, pb, run)
        else:
            final = _sc_final(pa, pb, run)
    return final[:_NUM_PLAYLISTS], final[_NUM_PLAYLISTS:_N]


# speedup vs baseline: 2.7063x; 2.7063x over previous
import functools

import jax
import jax.numpy as jnp
from jax import lax
from jax.experimental import pallas as pl
from jax.experimental.pallas import tpu as pltpu
from jax.experimental.pallas import tpu_sc as plsc

_NUM_PLAYLISTS = 2000
_NUM_TRACKS = 8000
_D = 128
_N = _NUM_PLAYLISTS + _NUM_TRACKS
_N_PAD = 10240
_E = 320000
_NLAYERS = 3

_CH = 64
_NWORKERS = 32
_NCHT = 160
_E_PAD = _NWORKERS * _NCHT * _CH
_ROWS_PER_TILE = _N_PAD // 16
_LANES = 16
_NBUF = 5

_mesh = plsc.VectorSubcoreMesh(core_axis_name="c", subcore_axis_name="s")

_scratch = (
    [pltpu.VMEM((_CH,), jnp.int32) for _ in range(_NBUF)]
    + [pltpu.VMEM((_CH,), jnp.int32) for _ in range(_NBUF)]
    + [pltpu.VMEM((_CH,), jnp.float32) for _ in range(_NBUF)]
    + [pltpu.VMEM((_CH, _D), jnp.float32) for _ in range(_NBUF)]
    + [pltpu.VMEM_SHARED((_N_PAD, _D), jnp.float32)]
    + [pltpu.SemaphoreType.DMA for _ in range(4 * _NBUF)]
)


@functools.partial(
    pl.kernel,
    out_type=[
        jax.ShapeDtypeStruct((_N_PAD, _D), jnp.float32),
        jax.ShapeDtypeStruct((_N_PAD, _D), jnp.float32),
    ],
    mesh=_mesh,
    scratch_types=_scratch,
)
def _sc_layer(emb, src1, dst1, w1, out0, out1, *refs):
    srcb = refs[0:_NBUF]
    dstb = refs[_NBUF:2 * _NBUF]
    wb = refs[2 * _NBUF:3 * _NBUF]
    rows = refs[3 * _NBUF:4 * _NBUF]
    acc = refs[4 * _NBUF]
    esemS = refs[4 * _NBUF + 1:4 * _NBUF + 1 + _NBUF]
    esemD = refs[4 * _NBUF + 1 + _NBUF:4 * _NBUF + 1 + 2 * _NBUF]
    gsem = refs[4 * _NBUF + 1 + 2 * _NBUF:4 * _NBUF + 1 + 3 * _NBUF]
    ssem = refs[4 * _NBUF + 1 + 3 * _NBUF:4 * _NBUF + 1 + 4 * _NBUF]

    c = lax.axis_index("c")
    s = lax.axis_index("s")
    wid = s * 2 + c
    ebase = wid * _NCHT * _CH


    def _scale(b):
        def body(eg, _):
            wvec = wb[b][pl.ds(eg * _LANES, _LANES)]
            for j in range(_LANES):
                wsc = wvec[j]
                e = eg * _LANES + j
                for d in range(_D // _LANES):
                    sl = pl.ds(d * _LANES, _LANES)
                    rows[b][e, sl] = rows[b][e, sl] * wsc
            return 0
        lax.fori_loop(0, _CH // _LANES, body, 0)

    def _ef_srcw(g, b):
        cb = ebase + g * _CH
        pltpu.async_copy(src1.at[pl.ds(cb, _CH)], srcb[b], esemS[b])
        pltpu.async_copy(w1.at[pl.ds(cb, _CH)], wb[b], esemS[b])

    def _efwait_srcw(g, b):
        cb = ebase + g * _CH
        pltpu.make_async_copy(src1.at[pl.ds(cb, _CH)], srcb[b], esemS[b]).wait()
        pltpu.make_async_copy(w1.at[pl.ds(cb, _CH)], wb[b], esemS[b]).wait()

    def _ef_dst(g, b):
        cb = ebase + g * _CH
        pltpu.async_copy(dst1.at[pl.ds(cb, _CH)], dstb[b], esemD[b])

    def _efwait_dst(g, b):
        cb = ebase + g * _CH
        pltpu.make_async_copy(dst1.at[pl.ds(cb, _CH)], dstb[b], esemD[b]).wait()

    for g in range(4):
        _ef_srcw(g, g)
    for g in range(3):
        _ef_dst(g, g)
    for g in range(3):
        _efwait_srcw(g, g)
        pltpu.async_copy(emb.at[srcb[g].at[pl.ds(0, 32)]], rows[g].at[pl.ds(0, 32)], gsem[g])
        pltpu.async_copy(emb.at[srcb[g].at[pl.ds(32, 32)]], rows[g].at[pl.ds(32, 32)], gsem[g])

    def _zero_row(i, _):
        for d in range(_D // _LANES):
            rows[4][i, pl.ds(d * _LANES, _LANES)] = jnp.zeros((_LANES,), jnp.float32)
        return 0
    lax.fori_loop(0, _CH, _zero_row, 0)
    rbase = s * _ROWS_PER_TILE
    for k in range(_ROWS_PER_TILE // _CH):
        pltpu.sync_copy(rows[4], acc.at[pl.ds(rbase + k * _CH, _CH)])
    plsc.subcore_barrier()

    def _iter(i, _):
        for b in range(_NBUF):
            g = i * _NBUF + b
            pltpu.make_async_copy(emb.at[srcb[b].at[pl.ds(0, 32)]], rows[b].at[pl.ds(0, 32)], gsem[b]).wait()
            pltpu.make_async_copy(emb.at[srcb[b].at[pl.ds(32, 32)]], rows[b].at[pl.ds(32, 32)], gsem[b]).wait()
            _scale(b)
            _efwait_dst(g, b)
            pltpu.async_copy(rows[b], acc.at[dstb[b]], ssem[b], add=True)

            bm2 = (b - 2) % _NBUF

            @pl.when(g >= 2)
            def _():
                pltpu.make_async_copy(rows[bm2], acc.at[dstb[bm2]], ssem[bm2]).wait()

            bp3 = (b + 3) % _NBUF
            bp4 = (b + 4) % _NBUF

            @pl.when(g + 3 < _NCHT)
            def _():
                _ef_dst(g + 3, bp3)

            @pl.when(g + 4 < _NCHT)
            def _():
                _ef_srcw(g + 4, bp4)

            @pl.when(g + 3 < _NCHT)
            def _():
                _efwait_srcw(g + 3, bp3)
                pltpu.async_copy(emb.at[srcb[bp3].at[pl.ds(0, 32)]], rows[bp3].at[pl.ds(0, 32)], gsem[bp3])
                pltpu.async_copy(emb.at[srcb[bp3].at[pl.ds(32, 32)]], rows[bp3].at[pl.ds(32, 32)], gsem[bp3])
        return 0
    lax.fori_loop(0, _NCHT // _NBUF, _iter, 0)
    pltpu.make_async_copy(rows[(_NCHT - 2) % _NBUF], acc.at[dstb[(_NCHT - 2) % _NBUF]], ssem[(_NCHT - 2) % _NBUF]).wait()
    pltpu.make_async_copy(rows[(_NCHT - 1) % _NBUF], acc.at[dstb[(_NCHT - 1) % _NBUF]], ssem[(_NCHT - 1) % _NBUF]).wait()
    plsc.subcore_barrier()

    plsc.subcore_barrier()

    for k in range(_ROWS_PER_TILE // 128):
        sl = pl.ds(rbase + k * 128, 128)

        @pl.when(c == 0)
        def _():
            pltpu.sync_copy(acc.at[sl], out0.at[sl])

        @pl.when(c == 1)
        def _():
            pltpu.sync_copy(acc.at[sl], out1.at[sl])


def _combine_body(pa_ref, pb_ref, run_ref, emb_ref, runo_ref):
    sm = pa_ref[...] + pb_ref[...]
    emb_ref[...] = sm
    runo_ref[...] = run_ref[...] + sm


def _final_body(pa_ref, pb_ref, run_ref, out_ref):
    out_ref[...] = (run_ref[...] + pa_ref[...] + pb_ref[...]) * 0.25


_bs = pl.BlockSpec((1024, _D), lambda i: (i, 0))
_sds = jax.ShapeDtypeStruct((_N_PAD, _D), jnp.float32)

_combine = pl.pallas_call(
    _combine_body, grid=(10,), in_specs=[_bs, _bs, _bs],
    out_specs=[_bs, _bs], out_shape=[_sds, _sds])

_final = pl.pallas_call(
    _final_body, grid=(10,), in_specs=[_bs, _bs, _bs],
    out_specs=_bs, out_shape=_sds)


def kernel(playlist_weight, track_weight, edge_index, edge_weight):
    emb0 = jnp.concatenate([playlist_weight, track_weight], axis=0)
    emb0 = jnp.pad(emb0, ((0, _N_PAD - _N), (0, 0)))
    src = edge_index[0].astype(jnp.int32)
    dst = edge_index[1].astype(jnp.int32)
    w = edge_weight.astype(jnp.float32)
    pad = _E_PAD - _E
    fill = _N + jnp.arange(pad, dtype=jnp.int32) % (_N_PAD - _N)
    src1 = jnp.concatenate([src, fill])
    dst1 = jnp.concatenate([dst, fill])
    w1 = jnp.concatenate([w, jnp.zeros((pad,), jnp.float32)])

    emb = emb0
    run = emb0
    final = None
    for layer in range(_NLAYERS):
        pa, pb = _sc_layer(emb, src1, dst1, w1)
        if layer < _NLAYERS - 1:
            emb, run = _combine(pa, pb, run)
        else:
            final = _final(pa, pb, run)
    return final[:_NUM_PLAYLISTS], final[_NUM_PLAYLISTS:_N]


# speedup vs baseline: 2.7376x; 1.0116x over previous
import functools

import jax
import jax.numpy as jnp
from jax import lax
from jax.experimental import pallas as pl
from jax.experimental.pallas import tpu as pltpu
from jax.experimental.pallas import tpu_sc as plsc

_NUM_PLAYLISTS = 2000
_NUM_TRACKS = 8000
_D = 128
_N = _NUM_PLAYLISTS + _NUM_TRACKS
_N_PAD = 10240
_E = 320000
_NLAYERS = 3

_CH = 64
_NWORKERS = 32
_NCHT = 160
_E_PAD = _NWORKERS * _NCHT * _CH
_ROWS_PER_TILE = _N_PAD // 16
_LANES = 16
_NBUF = 5

_mesh = plsc.VectorSubcoreMesh(core_axis_name="c", subcore_axis_name="s")

_scratch = (
    [pltpu.VMEM((_CH,), jnp.int32) for _ in range(_NBUF)]
    + [pltpu.VMEM((_CH,), jnp.int32) for _ in range(_NBUF)]
    + [pltpu.VMEM((_CH,), jnp.float32) for _ in range(_NBUF)]
    + [pltpu.VMEM((_CH, _D), jnp.float32) for _ in range(_NBUF)]
    + [pltpu.VMEM_SHARED((_N_PAD, _D), jnp.float32)]
    + [pltpu.SemaphoreType.DMA for _ in range(4 * _NBUF)]
)


@functools.partial(
    pl.kernel,
    out_type=[
        jax.ShapeDtypeStruct((_N_PAD, _D), jnp.float32),
        jax.ShapeDtypeStruct((_N_PAD, _D), jnp.float32),
    ],
    mesh=_mesh,
    scratch_types=_scratch,
)
def _sc_layer(emb, src1, dst1, w1, out0, out1, *refs):
    srcb = refs[0:_NBUF]
    dstb = refs[_NBUF:2 * _NBUF]
    wb = refs[2 * _NBUF:3 * _NBUF]
    rows = refs[3 * _NBUF:4 * _NBUF]
    acc = refs[4 * _NBUF]
    esemS = refs[4 * _NBUF + 1:4 * _NBUF + 1 + _NBUF]
    esemD = refs[4 * _NBUF + 1 + _NBUF:4 * _NBUF + 1 + 2 * _NBUF]
    gsem = refs[4 * _NBUF + 1 + 2 * _NBUF:4 * _NBUF + 1 + 3 * _NBUF]
    ssem = refs[4 * _NBUF + 1 + 3 * _NBUF:4 * _NBUF + 1 + 4 * _NBUF]

    c = lax.axis_index("c")
    s = lax.axis_index("s")
    wid = s * 2 + c
    ebase = wid * _NCHT * _CH


    def _scale(b):
        def body(eg, _):
            wvec = wb[b][pl.ds(eg * _LANES, _LANES)]
            for j in range(_LANES):
                wsc = wvec[j]
                e = eg * _LANES + j
                for d in range(_D // _LANES):
                    sl = pl.ds(d * _LANES, _LANES)
                    rows[b][e, sl] = rows[b][e, sl] * wsc
            return 0
        lax.fori_loop(0, _CH // _LANES, body, 0)

    def _ef_srcw(g, b):
        cb = ebase + g * _CH
        pltpu.async_copy(src1.at[pl.ds(cb, _CH)], srcb[b], esemS[b])
        pltpu.async_copy(w1.at[pl.ds(cb, _CH)], wb[b], esemS[b])

    def _efwait_srcw(g, b):
        cb = ebase + g * _CH
        pltpu.make_async_copy(src1.at[pl.ds(cb, _CH)], srcb[b], esemS[b]).wait()
        pltpu.make_async_copy(w1.at[pl.ds(cb, _CH)], wb[b], esemS[b]).wait()

    def _ef_dst(g, b):
        cb = ebase + g * _CH
        pltpu.async_copy(dst1.at[pl.ds(cb, _CH)], dstb[b], esemD[b])

    def _efwait_dst(g, b):
        cb = ebase + g * _CH
        pltpu.make_async_copy(dst1.at[pl.ds(cb, _CH)], dstb[b], esemD[b]).wait()

    for g in range(4):
        _ef_srcw(g, g)
    for g in range(3):
        _ef_dst(g, g)
    for g in range(3):
        _efwait_srcw(g, g)
        pltpu.async_copy(emb.at[srcb[g]], rows[g], gsem[g])

    def _zero_row(i, _):
        for d in range(_D // _LANES):
            rows[4][i, pl.ds(d * _LANES, _LANES)] = jnp.zeros((_LANES,), jnp.float32)
        return 0
    lax.fori_loop(0, _CH, _zero_row, 0)
    rbase = s * _ROWS_PER_TILE
    for k in range(_ROWS_PER_TILE // _CH):
        pltpu.sync_copy(rows[4], acc.at[pl.ds(rbase + k * _CH, _CH)])
    plsc.subcore_barrier()

    def _iter(i, _):
        for b in range(_NBUF):
            g = i * _NBUF + b
            pltpu.make_async_copy(emb.at[srcb[b]], rows[b], gsem[b]).wait()
            _scale(b)
            _efwait_dst(g, b)
            pltpu.async_copy(rows[b], acc.at[dstb[b]], ssem[b], add=True)

            bm2 = (b - 2) % _NBUF

            @pl.when(g >= 2)
            def _():
                pltpu.make_async_copy(rows[bm2], acc.at[dstb[bm2]], ssem[bm2]).wait()

            bp3 = (b + 3) % _NBUF
            bp4 = (b + 4) % _NBUF

            @pl.when(g + 3 < _NCHT)
            def _():
                _ef_dst(g + 3, bp3)

            @pl.when(g + 4 < _NCHT)
            def _():
                _ef_srcw(g + 4, bp4)

            @pl.when(g + 3 < _NCHT)
            def _():
                _efwait_srcw(g + 3, bp3)
                pltpu.async_copy(emb.at[srcb[bp3]], rows[bp3], gsem[bp3])
        return 0
    lax.fori_loop(0, _NCHT // _NBUF, _iter, 0)
    pltpu.make_async_copy(rows[(_NCHT - 2) % _NBUF], acc.at[dstb[(_NCHT - 2) % _NBUF]], ssem[(_NCHT - 2) % _NBUF]).wait()
    pltpu.make_async_copy(rows[(_NCHT - 1) % _NBUF], acc.at[dstb[(_NCHT - 1) % _NBUF]], ssem[(_NCHT - 1) % _NBUF]).wait()
    plsc.subcore_barrier()

    plsc.subcore_barrier()

    for k in range(_ROWS_PER_TILE // 128):
        sl = pl.ds(rbase + k * 128, 128)

        @pl.when(c == 0)
        def _():
            pltpu.sync_copy(acc.at[sl], out0.at[sl])

        @pl.when(c == 1)
        def _():
            pltpu.sync_copy(acc.at[sl], out1.at[sl])


def _emb_body(pa_ref, pb_ref, emb_ref):
    emb_ref[...] = pa_ref[...] + pb_ref[...]


def _run_body(pa_ref, pb_ref, run_ref, runo_ref):
    runo_ref[...] = run_ref[...] + pa_ref[...] + pb_ref[...]


def _final_body(pa_ref, pb_ref, run_ref, out_ref):
    out_ref[...] = (run_ref[...] + pa_ref[...] + pb_ref[...]) * 0.25


_bs = pl.BlockSpec((1024, _D), lambda i: (i, 0))
_sds = jax.ShapeDtypeStruct((_N_PAD, _D), jnp.float32)

_emb_k = pl.pallas_call(
    _emb_body, grid=(10,), in_specs=[_bs, _bs],
    out_specs=_bs, out_shape=_sds)

_run_k = pl.pallas_call(
    _run_body, grid=(10,), in_specs=[_bs, _bs, _bs],
    out_specs=_bs, out_shape=_sds)

_final = pl.pallas_call(
    _final_body, grid=(10,), in_specs=[_bs, _bs, _bs],
    out_specs=_bs, out_shape=_sds)


def kernel(playlist_weight, track_weight, edge_index, edge_weight):
    emb0 = jnp.concatenate([playlist_weight, track_weight], axis=0)
    emb0 = jnp.pad(emb0, ((0, _N_PAD - _N), (0, 0)))
    src = edge_index[0].astype(jnp.int32)
    dst = edge_index[1].astype(jnp.int32)
    w = edge_weight.astype(jnp.float32)
    pad = _E_PAD - _E
    fill = _N + jnp.arange(pad, dtype=jnp.int32) % (_N_PAD - _N)
    src1 = jnp.concatenate([src, fill])
    dst1 = jnp.concatenate([dst, fill])
    w1 = jnp.concatenate([w, jnp.zeros((pad,), jnp.float32)])

    emb = emb0
    run = emb0
    final = None
    for layer in range(_NLAYERS):
        pa, pb = _sc_layer(emb, src1, dst1, w1)
        if layer < _NLAYERS - 1:
            emb = _emb_k(pa, pb)
            run = _run_k(pa, pb, run)
        else:
            final = _final(pa, pb, run)
    return final[:_NUM_PLAYLISTS], final[_NUM_PLAYLISTS:_N]


# speedup vs baseline: 2.7416x; 1.0015x over previous
import functools

import jax
import jax.numpy as jnp
from jax import lax
from jax.experimental import pallas as pl
from jax.experimental.pallas import tpu as pltpu
from jax.experimental.pallas import tpu_sc as plsc

_NUM_PLAYLISTS = 2000
_NUM_TRACKS = 8000
_D = 128
_N = _NUM_PLAYLISTS + _NUM_TRACKS
_N_PAD = 10240
_E = 320000
_NLAYERS = 3

_CH = 64
_NWORKERS = 32
_NCHT = 160
_E_PAD = _NWORKERS * _NCHT * _CH
_ROWS_PER_TILE = _N_PAD // 16
_LANES = 16
_NBUF = 5

_mesh = plsc.VectorSubcoreMesh(core_axis_name="c", subcore_axis_name="s")

_scratch = (
    [pltpu.VMEM((_CH,), jnp.int32) for _ in range(_NBUF)]
    + [pltpu.VMEM((_CH,), jnp.int32) for _ in range(_NBUF)]
    + [pltpu.VMEM((_CH,), jnp.float32) for _ in range(_NBUF)]
    + [pltpu.VMEM((_CH, _D), jnp.float32) for _ in range(_NBUF)]
    + [pltpu.VMEM_SHARED((_N_PAD, _D), jnp.float32)]
    + [pltpu.SemaphoreType.DMA for _ in range(4 * _NBUF)]
)


@functools.partial(
    pl.kernel,
    out_type=[
        jax.ShapeDtypeStruct((_N_PAD, _D), jnp.float32),
        jax.ShapeDtypeStruct((_N_PAD, _D), jnp.float32),
    ],
    mesh=_mesh,
    scratch_types=_scratch,
)
def _sc_layer(emb, src1, dst1, w1, out0, out1, *refs):
    srcb = refs[0:_NBUF]
    dstb = refs[_NBUF:2 * _NBUF]
    wb = refs[2 * _NBUF:3 * _NBUF]
    rows = refs[3 * _NBUF:4 * _NBUF]
    acc = refs[4 * _NBUF]
    esemS = refs[4 * _NBUF + 1:4 * _NBUF + 1 + _NBUF]
    esemD = refs[4 * _NBUF + 1 + _NBUF:4 * _NBUF + 1 + 2 * _NBUF]
    gsem = refs[4 * _NBUF + 1 + 2 * _NBUF:4 * _NBUF + 1 + 3 * _NBUF]
    ssem = refs[4 * _NBUF + 1 + 3 * _NBUF:4 * _NBUF + 1 + 4 * _NBUF]

    c = lax.axis_index("c")
    s = lax.axis_index("s")
    wid = s * 2 + c
    ebase = wid * _NCHT * _CH


    def _scale(b):
        def body(eg, _):
            wvec = wb[b][pl.ds(eg * _LANES, _LANES)]
            for j in range(_LANES):
                wsc = wvec[j]
                e = eg * _LANES + j
                for d in range(_D // _LANES):
                    sl = pl.ds(d * _LANES, _LANES)
                    rows[b][e, sl] = rows[b][e, sl] * wsc
            return 0
        lax.fori_loop(0, _CH // _LANES, body, 0)

    def _ef_srcw(g, b):
        cb = ebase + g * _CH
        pltpu.async_copy(src1.at[pl.ds(cb, _CH)], srcb[b], esemS[b])
        pltpu.async_copy(w1.at[pl.ds(cb, _CH)], wb[b], esemS[b])

    def _efwait_srcw(g, b):
        cb = ebase + g * _CH
        pltpu.make_async_copy(src1.at[pl.ds(cb, _CH)], srcb[b], esemS[b]).wait()
        pltpu.make_async_copy(w1.at[pl.ds(cb, _CH)], wb[b], esemS[b]).wait()

    def _ef_dst(g, b):
        cb = ebase + g * _CH
        pltpu.async_copy(dst1.at[pl.ds(cb, _CH)], dstb[b], esemD[b])

    def _efwait_dst(g, b):
        cb = ebase + g * _CH
        pltpu.make_async_copy(dst1.at[pl.ds(cb, _CH)], dstb[b], esemD[b]).wait()

    for g in range(4):
        _ef_srcw(g, g)
    for g in range(3):
        _ef_dst(g, g)
    for g in range(3):
        _efwait_srcw(g, g)
        pltpu.async_copy(emb.at[srcb[g]], rows[g], gsem[g])

    def _zero_row(i, _):
        for d in range(_D // _LANES):
            rows[4][i, pl.ds(d * _LANES, _LANES)] = jnp.zeros((_LANES,), jnp.float32)
        return 0
    lax.fori_loop(0, _CH, _zero_row, 0)
    rbase = s * _ROWS_PER_TILE
    for k in range(_ROWS_PER_TILE // _CH):
        pltpu.sync_copy(rows[4], acc.at[pl.ds(rbase + k * _CH, _CH)])
    plsc.subcore_barrier()

    def _iter(i, _):
        for b in range(_NBUF):
            g = i * _NBUF + b
            pltpu.make_async_copy(emb.at[srcb[b]], rows[b], gsem[b]).wait()
            _scale(b)
            _efwait_dst(g, b)
            pltpu.async_copy(rows[b], acc.at[dstb[b]], ssem[b], add=True)

            bm2 = (b - 2) % _NBUF

            @pl.when(g >= 2)
            def _():
                pltpu.make_async_copy(rows[bm2], acc.at[dstb[bm2]], ssem[bm2]).wait()

            bp3 = (b + 3) % _NBUF
            bp4 = (b + 4) % _NBUF

            @pl.when(g + 3 < _NCHT)
            def _():
                _ef_dst(g + 3, bp3)

            @pl.when(g + 4 < _NCHT)
            def _():
                _ef_srcw(g + 4, bp4)

            @pl.when(g + 3 < _NCHT)
            def _():
                _efwait_srcw(g + 3, bp3)
                pltpu.async_copy(emb.at[srcb[bp3]], rows[bp3], gsem[bp3])
        return 0
    lax.fori_loop(0, _NCHT // _NBUF, _iter, 0)
    pltpu.make_async_copy(rows[(_NCHT - 2) % _NBUF], acc.at[dstb[(_NCHT - 2) % _NBUF]], ssem[(_NCHT - 2) % _NBUF]).wait()
    pltpu.make_async_copy(rows[(_NCHT - 1) % _NBUF], acc.at[dstb[(_NCHT - 1) % _NBUF]], ssem[(_NCHT - 1) % _NBUF]).wait()
    plsc.subcore_barrier()

    plsc.subcore_barrier()

    for k in range(_ROWS_PER_TILE // 128):
        sl = pl.ds(rbase + k * 128, 128)

        @pl.when(c == 0)
        def _():
            pltpu.sync_copy(acc.at[sl], out0.at[sl])

        @pl.when(c == 1)
        def _():
            pltpu.sync_copy(acc.at[sl], out1.at[sl])


def _combine_body(pa_ref, pb_ref, run_ref, emb_ref, runo_ref):
    sm = pa_ref[...] + pb_ref[...]
    emb_ref[...] = sm
    runo_ref[...] = run_ref[...] + sm


def _final_body(pa_ref, pb_ref, run_ref, out_ref):
    out_ref[...] = (run_ref[...] + pa_ref[...] + pb_ref[...]) * 0.25


_bs = pl.BlockSpec((1024, _D), lambda i: (i, 0))
_sds = jax.ShapeDtypeStruct((_N_PAD, _D), jnp.float32)

_combine = pl.pallas_call(
    _combine_body, grid=(10,), in_specs=[_bs, _bs, _bs],
    out_specs=[_bs, _bs], out_shape=[_sds, _sds])

_final = pl.pallas_call(
    _final_body, grid=(10,), in_specs=[_bs, _bs, _bs],
    out_specs=_bs, out_shape=_sds)


def kernel(playlist_weight, track_weight, edge_index, edge_weight):
    emb0 = jnp.concatenate([playlist_weight, track_weight], axis=0)
    emb0 = jnp.pad(emb0, ((0, _N_PAD - _N), (0, 0)))
    src = edge_index[0].astype(jnp.int32)
    dst = edge_index[1].astype(jnp.int32)
    w = edge_weight.astype(jnp.float32)
    pad = _E_PAD - _E
    fill = _N + jnp.arange(pad, dtype=jnp.int32) % (_N_PAD - _N)
    src1 = jnp.concatenate([src, fill])
    dst1 = jnp.concatenate([dst, fill])
    w1 = jnp.concatenate([w, jnp.zeros((pad,), jnp.float32)])

    emb = emb0
    run = emb0
    final = None
    for layer in range(_NLAYERS):
        pa, pb = _sc_layer(emb, src1, dst1, w1)
        if layer < _NLAYERS - 1:
            emb, run = _combine(pa, pb, run)
        else:
            final = _final(pa, pb, run)
    return final[:_NUM_PLAYLISTS], final[_NUM_PLAYLISTS:_N]
